# baseline (speedup 1.0000x reference)
import functools

import jax
import jax.numpy as jnp
from jax.experimental import pallas as pl
from jax.experimental.pallas import tpu as pltpu

N_EXP = 4
NP = 8192
NL = 2048
DP = 64
DL = 64
DO = 64
DIN = DP + DL
EPS = 1e-5

BLK = 1024
NR = NP // BLK


def _body(am_ref, px_ref, lx_ref,
          Wpl_ref, bpl_ref, gpl_ref, bepl_ref,
          Wlp_ref, blp_ref, glp_ref, belp_ref,
          wp_ref, bp_ref, wl_ref, bl_ref,
          px_out_ref, lx_out_ref,
          z2_ref, x1T_ref, Li_ref):
    i = pl.program_id(0)
    r = pl.program_id(1)
    nr = pl.num_programs(1)

    am = am_ref[0]
    pxb = px_ref[...]

    @pl.when(r == 0)
    def _():
        Li_ref[...] = jnp.dot(lx_ref[...], Wpl_ref[0][:DL],
                              preferred_element_type=jnp.float32)

    z2 = (jnp.dot(am, Li_ref[...], preferred_element_type=jnp.float32)
          + jnp.dot(pxb, Wpl_ref[0][DL:], preferred_element_type=jnp.float32)
          + bpl_ref[0])
    z2_ref[pl.ds(r * BLK, BLK), :] = z2

    pxw = jnp.dot(pxb, Wlp_ref[0][:DP],
                  preferred_element_type=jnp.float32)
    partT = jax.lax.dot_general(pxw, am, (((0,), (0,)), ((), ())),
                                preferred_element_type=jnp.float32)

    @pl.when(r == 0)
    def _():
        x1T_ref[...] = partT

    @pl.when(r != 0)
    def _():
        x1T_ref[...] = x1T_ref[...] + partT

    @pl.when(r == nr - 1)
    def _():
        z1 = (x1T_ref[...].T
              + jnp.dot(lx_ref[...], Wlp_ref[0][DP:],
                        preferred_element_type=jnp.float32)
              + blp_ref[0])
        m1 = jnp.mean(z1, axis=0, keepdims=True)
        v1 = jnp.mean(z1 * z1, axis=0, keepdims=True) - m1 * m1
        s1 = glp_ref[0] * jax.lax.rsqrt(v1 + EPS)
        p1 = jnp.maximum(z1 * s1 + (belp_ref[0] - m1 * s1), 0.0)
        contrib_l = wl_ref[0] * p1

        @pl.when(i == 0)
        def _():
            lx_out_ref[...] = contrib_l + bl_ref[0]

        @pl.when(i != 0)
        def _():
            lx_out_ref[...] = lx_out_ref[...] + contrib_l

        zb = z2_ref[...]
        m2 = jnp.mean(zb, axis=0, keepdims=True)
        zc = zb - m2
        v2 = jnp.mean(zc * zc, axis=0, keepdims=True)
        s2 = gpl_ref[0] * jax.lax.rsqrt(v2 + EPS)
        p2 = jnp.maximum(zb * s2 + (bepl_ref[0] - m2 * s2), 0.0)
        contrib_p = wp_ref[0] * p2

        @pl.when(i == 0)
        def _():
            px_out_ref[...] = contrib_p + bp_ref[0]

        @pl.when(i != 0)
        def _():
            px_out_ref[...] = px_out_ref[...] + contrib_p


@functools.partial(jax.jit, static_argnames=("interpret",))
def _run(px, lx, pl_mat_list, W_pl, b_pl, g_pl, be_pl,
         W_lp, b_lp, g_lp, be_lp, wp3, bp3, wl3, bl3, interpret=False):
    term_blk = lambda i, r: (i, 0, 0)
    const2 = lambda i, r: (0, 0)
    const3 = lambda i, r: (0, 0, 0)
    vec_spec = pl.BlockSpec((1, 1, DO), term_blk)
    sca_spec = pl.BlockSpec((1, 1, 1), term_blk)
    return pl.pallas_call(
        _body,
        grid=(N_EXP, NR),
        in_specs=[
            pl.BlockSpec((1, BLK, NL), lambda i, r: (i, r, 0)),
            pl.BlockSpec((BLK, DP), lambda i, r: (r, 0)),
            pl.BlockSpec((NL, DL), const2),
            pl.BlockSpec((1, DIN, DO), term_blk),
            vec_spec,
            vec_spec,
            vec_spec,
            pl.BlockSpec((1, DIN, DO), term_blk),
            vec_spec,
            vec_spec,
            vec_spec,
            sca_spec,
            pl.BlockSpec((1, 1, 1), const3),
            sca_spec,
            pl.BlockSpec((1, 1, 1), const3),
        ],
        out_specs=[
            pl.BlockSpec((NP, DO), const2),
            pl.BlockSpec((NL, DO), const2),
        ],
        out_shape=[
            jax.ShapeDtypeStruct((NP, DO), jnp.float32),
            jax.ShapeDtypeStruct((NL, DO), jnp.float32),
        ],
        scratch_shapes=[
            pltpu.VMEM((NP, DO), jnp.float32),
            pltpu.VMEM((DO, NL), jnp.float32),
            pltpu.VMEM((NL, DO), jnp.float32),
        ],
        compiler_params=pltpu.CompilerParams(
            dimension_semantics=("arbitrary", "arbitrary"),
        ),
        interpret=interpret,
    )(pl_mat_list, px, lx, W_pl, b_pl, g_pl, be_pl,
      W_lp, b_lp, g_lp, be_lp, wp3, bp3, wl3, bl3)


def kernel(px, lx, pl_mat_list, W_pl, b_pl, g_pl, be_pl,
           W_lp, b_lp, g_lp, be_lp, wp, bp, wl, bl):
    v = lambda a: a.reshape(N_EXP, 1, DO)
    px_out, lx_out = _run(px, lx, pl_mat_list,
                          W_pl, v(b_pl), v(g_pl), v(be_pl),
                          W_lp, v(b_lp), v(g_lp), v(be_lp),
                          wp.reshape(N_EXP, 1, 1), bp.reshape(1, 1, 1),
                          wl.reshape(N_EXP, 1, 1), bl.reshape(1, 1, 1))
    return (px_out, lx_out)


# speedup vs baseline: 1.0023x; 1.0023x over previous
import functools

import jax
import jax.numpy as jnp
from jax.experimental import pallas as pl
from jax.experimental.pallas import tpu as pltpu

N_EXP = 4
NP = 8192
NL = 2048
DP = 64
DL = 64
DO = 64
DIN = DP + DL
EPS = 1e-5

BLK = 1024
NR = NP // BLK


def _body(am_ref, px_ref, lx_ref,
          Wpl_ref, bpl_ref, gpl_ref, bepl_ref,
          Wlp_ref, blp_ref, glp_ref, belp_ref,
          wp_ref, bp_ref, wl_ref, bl_ref,
          px_out_ref, lx_out_ref,
          z2_ref, x1T_ref, Li_ref):
    i = pl.program_id(0)
    r = pl.program_id(1)
    nr = pl.num_programs(1)

    am = am_ref[...]
    pxb = px_ref[...]

    @pl.when(r == 0)
    def _():
        Li_ref[...] = jnp.dot(lx_ref[...], Wpl_ref[0][:DL],
                              preferred_element_type=jnp.float32)

    z2 = (jnp.dot(am, Li_ref[...], preferred_element_type=jnp.float32)
          + jnp.dot(pxb, Wpl_ref[0][DL:], preferred_element_type=jnp.float32)
          + bpl_ref[0])
    z2_ref[pl.ds(r * BLK, BLK), :] = z2

    pxw = jnp.dot(pxb, Wlp_ref[0][:DP],
                  preferred_element_type=jnp.float32)
    partT = jax.lax.dot_general(pxw, am, (((0,), (0,)), ((), ())),
                                preferred_element_type=jnp.float32)

    @pl.when(r == 0)
    def _():
        x1T_ref[...] = partT

    @pl.when(r != 0)
    def _():
        x1T_ref[...] = x1T_ref[...] + partT

    @pl.when(r == nr - 1)
    def _():
        z1 = (x1T_ref[...].T
              + jnp.dot(lx_ref[...], Wlp_ref[0][DP:],
                        preferred_element_type=jnp.float32)
              + blp_ref[0])
        m1 = jnp.mean(z1, axis=0, keepdims=True)
        v1 = jnp.mean(z1 * z1, axis=0, keepdims=True) - m1 * m1
        s1 = glp_ref[0] * jax.lax.rsqrt(v1 + EPS)
        p1 = jnp.maximum(z1 * s1 + (belp_ref[0] - m1 * s1), 0.0)
        contrib_l = wl_ref[0] * p1

        @pl.when(i == 0)
        def _():
            lx_out_ref[...] = contrib_l + bl_ref[0]

        @pl.when(i != 0)
        def _():
            lx_out_ref[...] = lx_out_ref[...] + contrib_l

        zb = z2_ref[...]
        m2 = jnp.mean(zb, axis=0, keepdims=True)
        zc = zb - m2
        v2 = jnp.mean(zc * zc, axis=0, keepdims=True)
        s2 = gpl_ref[0] * jax.lax.rsqrt(v2 + EPS)
        p2 = jnp.maximum(zb * s2 + (bepl_ref[0] - m2 * s2), 0.0)
        contrib_p = wp_ref[0] * p2

        @pl.when(i == 0)
        def _():
            px_out_ref[...] = contrib_p + bp_ref[0]

        @pl.when(i != 0)
        def _():
            px_out_ref[...] = px_out_ref[...] + contrib_p


@functools.partial(jax.jit, static_argnames=("interpret",))
def _run(px, lx, pl_mat_list, W_pl, b_pl, g_pl, be_pl,
         W_lp, b_lp, g_lp, be_lp, wp3, bp3, wl3, bl3, interpret=False):
    term_blk = lambda i, r: (i, 0, 0)
    const2 = lambda i, r: (0, 0)
    const3 = lambda i, r: (0, 0, 0)
    vec_spec = pl.BlockSpec((1, 1, DO), term_blk)
    sca_spec = pl.BlockSpec((1, 1, 1), term_blk)
    return pl.pallas_call(
        _body,
        grid=(N_EXP, NR),
        in_specs=[
            pl.BlockSpec((BLK, NL), lambda i, r: (i * NR + r, 0)),
            pl.BlockSpec((BLK, DP), lambda i, r: (r, 0)),
            pl.BlockSpec((NL, DL), const2),
            pl.BlockSpec((1, DIN, DO), term_blk),
            vec_spec,
            vec_spec,
            vec_spec,
            pl.BlockSpec((1, DIN, DO), term_blk),
            vec_spec,
            vec_spec,
            vec_spec,
            sca_spec,
            pl.BlockSpec((1, 1, 1), const3),
            sca_spec,
            pl.BlockSpec((1, 1, 1), const3),
        ],
        out_specs=[
            pl.BlockSpec((NP, DO), const2),
            pl.BlockSpec((NL, DO), const2),
        ],
        out_shape=[
            jax.ShapeDtypeStruct((NP, DO), jnp.float32),
            jax.ShapeDtypeStruct((NL, DO), jnp.float32),
        ],
        scratch_shapes=[
            pltpu.VMEM((NP, DO), jnp.float32),
            pltpu.VMEM((DO, NL), jnp.float32),
            pltpu.VMEM((NL, DO), jnp.float32),
        ],
        compiler_params=pltpu.CompilerParams(
            dimension_semantics=("arbitrary", "arbitrary"),
        ),
        interpret=interpret,
    )(pl_mat_list.reshape(N_EXP * NP, NL), px, lx, W_pl, b_pl, g_pl, be_pl,
      W_lp, b_lp, g_lp, be_lp, wp3, bp3, wl3, bl3)


def kernel(px, lx, pl_mat_list, W_pl, b_pl, g_pl, be_pl,
           W_lp, b_lp, g_lp, be_lp, wp, bp, wl, bl):
    v = lambda a: a.reshape(N_EXP, 1, DO)
    px_out, lx_out = _run(px, lx, pl_mat_list,
                          W_pl, v(b_pl), v(g_pl), v(be_pl),
                          W_lp, v(b_lp), v(g_lp), v(be_lp),
                          wp.reshape(N_EXP, 1, 1), bp.reshape(1, 1, 1),
                          wl.reshape(N_EXP, 1, 1), bl.reshape(1, 1, 1))
    return (px_out, lx_out)


# speedup vs baseline: 1.0173x; 1.0150x over previous
import functools

import jax
import jax.numpy as jnp
from jax.experimental import pallas as pl
from jax.experimental.pallas import tpu as pltpu

N_EXP = 4
NP = 8192
NL = 2048
DP = 64
DL = 64
DO = 64
DIN = DP + DL
EPS = 1e-5

BLK = 1024
NR = NP // BLK
FCHUNK = 1024


def _body(am_ref, px_ref, lx_ref,
          Wpl_ref, bpl_ref, gpl_ref, bepl_ref,
          Wlp_ref, blp_ref, glp_ref, belp_ref,
          wp_ref, bp_ref, wl_ref, bl_ref,
          px_out_ref, lx_out_ref,
          z2_ref, x1T_ref, Li_ref, s_ref, q_ref):
    i = pl.program_id(0)
    r = pl.program_id(1)
    nr = pl.num_programs(1)

    am = am_ref[...]
    pxb = px_ref[...]

    @pl.when(r == 0)
    def _():
        Li_ref[...] = jnp.dot(lx_ref[...], Wpl_ref[0][:DL],
                              preferred_element_type=jnp.float32)

    z2 = (jnp.dot(am, Li_ref[...], preferred_element_type=jnp.float32)
          + jnp.dot(pxb, Wpl_ref[0][DL:], preferred_element_type=jnp.float32)
          + bpl_ref[0])
    z2_ref[pl.ds(i * NP + r * BLK, BLK), :] = z2

    s_blk = jnp.sum(z2, axis=0, keepdims=True)
    q_blk = jnp.sum(z2 * z2, axis=0, keepdims=True)

    @pl.when(r == 0)
    def _():
        s_ref[pl.ds(i, 1), :] = s_blk
        q_ref[pl.ds(i, 1), :] = q_blk

    @pl.when(r != 0)
    def _():
        s_ref[pl.ds(i, 1), :] = s_ref[pl.ds(i, 1), :] + s_blk
        q_ref[pl.ds(i, 1), :] = q_ref[pl.ds(i, 1), :] + q_blk

    pxw = jnp.dot(pxb, Wlp_ref[0][:DP], preferred_element_type=jnp.float32)
    partT = jax.lax.dot_general(pxw, am, (((0,), (0,)), ((), ())),
                                preferred_element_type=jnp.float32)

    @pl.when(r == 0)
    def _():
        x1T_ref[...] = partT

    @pl.when(r != 0)
    def _():
        x1T_ref[...] = x1T_ref[...] + partT

    @pl.when(r == nr - 1)
    def _():
        z1 = (x1T_ref[...].T
              + jnp.dot(lx_ref[...], Wlp_ref[0][DP:],
                        preferred_element_type=jnp.float32)
              + blp_ref[0])
        m1 = jnp.mean(z1, axis=0, keepdims=True)
        v1 = jnp.mean(z1 * z1, axis=0, keepdims=True) - m1 * m1
        s1 = glp_ref[0] * jax.lax.rsqrt(v1 + EPS)
        p1 = jnp.maximum(z1 * s1 + (belp_ref[0] - m1 * s1), 0.0)
        contrib_l = wl_ref[0] * p1

        @pl.when(i == 0)
        def _():
            lx_out_ref[...] = contrib_l + bl_ref[0]

        @pl.when(i != 0)
        def _():
            lx_out_ref[...] = lx_out_ref[...] + contrib_l

    @pl.when((i == pl.num_programs(0) - 1) & (r == nr - 1))
    def _():
        scales = []
        shifts = []
        for t in range(N_EXP):
            m = s_ref[t:t + 1, :] * (1.0 / NP)
            v = q_ref[t:t + 1, :] * (1.0 / NP) - m * m
            sc = gpl_ref[t] * jax.lax.rsqrt(v + EPS)
            scales.append(sc)
            shifts.append(bepl_ref[t] - m * sc)

        def chunk(c, carry):
            base = c * FCHUNK
            acc = jnp.zeros((FCHUNK, DO), jnp.float32) + bp_ref[0]
            for t in range(N_EXP):
                z = z2_ref[pl.ds(t * NP + base, FCHUNK), :]
                p = jnp.maximum(z * scales[t] + shifts[t], 0.0)
                acc = acc + wp_ref[t] * p
            px_out_ref[pl.ds(base, FCHUNK), :] = acc
            return carry

        jax.lax.fori_loop(0, NP // FCHUNK, chunk, 0, unroll=False)


@functools.partial(jax.jit, static_argnames=("interpret",))
def _run(px, lx, pl_mat_list, W_pl, b_pl, g_pl, be_pl,
         W_lp, b_lp, g_lp, be_lp, wp3, bp3, wl3, bl3, interpret=False):
    term_blk = lambda i, r: (i, 0, 0)
    const2 = lambda i, r: (0, 0)
    const3 = lambda i, r: (0, 0, 0)
    vec_spec = pl.BlockSpec((1, 1, DO), term_blk)
    full_vec_spec = pl.BlockSpec((N_EXP, 1, DO), const3)
    return pl.pallas_call(
        _body,
        grid=(N_EXP, NR),
        in_specs=[
            pl.BlockSpec((BLK, NL), lambda i, r: (i * NR + r, 0)),
            pl.BlockSpec((BLK, DP), lambda i, r: (r, 0)),
            pl.BlockSpec((NL, DL), const2),
            pl.BlockSpec((1, DIN, DO), term_blk),
            vec_spec,
            full_vec_spec,
            full_vec_spec,
            pl.BlockSpec((1, DIN, DO), term_blk),
            vec_spec,
            vec_spec,
            vec_spec,
            pl.BlockSpec((N_EXP, 1, 1), const3),
            pl.BlockSpec((1, 1, 1), const3),
            pl.BlockSpec((1, 1, 1), term_blk),
            pl.BlockSpec((1, 1, 1), const3),
        ],
        out_specs=[
            pl.BlockSpec((NP, DO), const2),
            pl.BlockSpec((NL, DO), const2),
        ],
        out_shape=[
            jax.ShapeDtypeStruct((NP, DO), jnp.float32),
            jax.ShapeDtypeStruct((NL, DO), jnp.float32),
        ],
        scratch_shapes=[
            pltpu.VMEM((N_EXP * NP, DO), jnp.float32),
            pltpu.VMEM((DO, NL), jnp.float32),
            pltpu.VMEM((NL, DO), jnp.float32),
            pltpu.VMEM((8, DO), jnp.float32),
            pltpu.VMEM((8, DO), jnp.float32),
        ],
        compiler_params=pltpu.CompilerParams(
            dimension_semantics=("arbitrary", "arbitrary"),
        ),
        interpret=interpret,
    )(pl_mat_list.reshape(N_EXP * NP, NL), px, lx, W_pl, b_pl, g_pl, be_pl,
      W_lp, b_lp, g_lp, be_lp, wp3, bp3, wl3, bl3)


def kernel(px, lx, pl_mat_list, W_pl, b_pl, g_pl, be_pl,
           W_lp, b_lp, g_lp, be_lp, wp, bp, wl, bl):
    v = lambda a: a.reshape(N_EXP, 1, DO)
    px_out, lx_out = _run(px, lx, pl_mat_list,
                          W_pl, v(b_pl), v(g_pl), v(be_pl),
                          W_lp, v(b_lp), v(g_lp), v(be_lp),
                          wp.reshape(N_EXP, 1, 1), bp.reshape(1, 1, 1),
                          wl.reshape(N_EXP, 1, 1), bl.reshape(1, 1, 1))
    return (px_out, lx_out)


# speedup vs baseline: 1.0288x; 1.0113x over previous
import functools

import jax
import jax.numpy as jnp
from jax.experimental import pallas as pl
from jax.experimental.pallas import tpu as pltpu

N_EXP = 4
NP = 8192
NL = 2048
DP = 64
DL = 64
DO = 64
DIN = DP + DL
EPS = 1e-5

BLK = 1024
NR = NP // BLK
FCHUNK = 1024


def _body(am_hbm_ref, px_ref, lx_ref,
          Wpl_ref, bpl_ref, gpl_ref, bepl_ref,
          Wlp_ref, blp_ref, glp_ref, belp_ref,
          wp_ref, bp_ref, wl_ref, bl_ref,
          px_out_ref, lx_out_ref,
          z2_ref, x1T_ref, Li_ref, s_ref, q_ref, am_buf, dma_sem):
    i = pl.program_id(0)
    r = pl.program_id(1)
    nr = pl.num_programs(1)
    k = i * nr + r
    nk = pl.num_programs(0) * nr

    def start_copy(kk):
        slot = jax.lax.rem(kk, 2)
        pltpu.make_async_copy(
            am_hbm_ref.at[pl.ds(kk * BLK, BLK), :],
            am_buf.at[slot], dma_sem.at[slot]).start()

    @pl.when(k == 0)
    def _():
        start_copy(0)

    @pl.when(k + 1 < nk)
    def _():
        start_copy(k + 1)

    slot = jax.lax.rem(k, 2)
    pltpu.make_async_copy(
        am_hbm_ref.at[pl.ds(k * BLK, BLK), :],
        am_buf.at[slot], dma_sem.at[slot]).wait()

    am = am_buf[slot]
    pxb = px_ref[...]

    @pl.when(r == 0)
    def _():
        Li_ref[...] = jnp.dot(lx_ref[...], Wpl_ref[0][:DL],
                              preferred_element_type=jnp.float32)

    z2 = (jnp.dot(am, Li_ref[...], preferred_element_type=jnp.float32)
          + jnp.dot(pxb, Wpl_ref[0][DL:], preferred_element_type=jnp.float32)
          + bpl_ref[0])
    z2_ref[pl.ds(i * NP + r * BLK, BLK), :] = z2

    s_blk = jnp.sum(z2, axis=0, keepdims=True)
    q_blk = jnp.sum(z2 * z2, axis=0, keepdims=True)

    @pl.when(r == 0)
    def _():
        s_ref[pl.ds(i, 1), :] = s_blk
        q_ref[pl.ds(i, 1), :] = q_blk

    @pl.when(r != 0)
    def _():
        s_ref[pl.ds(i, 1), :] = s_ref[pl.ds(i, 1), :] + s_blk
        q_ref[pl.ds(i, 1), :] = q_ref[pl.ds(i, 1), :] + q_blk

    pxw = jnp.dot(pxb, Wlp_ref[0][:DP], preferred_element_type=jnp.float32)
    partT = jax.lax.dot_general(pxw, am, (((0,), (0,)), ((), ())),
                                preferred_element_type=jnp.float32)

    @pl.when(r == 0)
    def _():
        x1T_ref[...] = partT

    @pl.when(r != 0)
    def _():
        x1T_ref[...] = x1T_ref[...] + partT

    @pl.when(r == nr - 1)
    def _():
        z1 = (x1T_ref[...].T
              + jnp.dot(lx_ref[...], Wlp_ref[0][DP:],
                        preferred_element_type=jnp.float32)
              + blp_ref[0])
        m1 = jnp.mean(z1, axis=0, keepdims=True)
        v1 = jnp.mean(z1 * z1, axis=0, keepdims=True) - m1 * m1
        s1 = glp_ref[0] * jax.lax.rsqrt(v1 + EPS)
        p1 = jnp.maximum(z1 * s1 + (belp_ref[0] - m1 * s1), 0.0)
        contrib_l = wl_ref[0] * p1

        @pl.when(i == 0)
        def _():
            lx_out_ref[...] = contrib_l + bl_ref[0]

        @pl.when(i != 0)
        def _():
            lx_out_ref[...] = lx_out_ref[...] + contrib_l

    @pl.when((i == pl.num_programs(0) - 1) & (r == nr - 1))
    def _():
        scales = []
        shifts = []
        for t in range(N_EXP):
            m = s_ref[t:t + 1, :] * (1.0 / NP)
            v = q_ref[t:t + 1, :] * (1.0 / NP) - m * m
            sc = gpl_ref[t] * jax.lax.rsqrt(v + EPS)
            scales.append(sc)
            shifts.append(bepl_ref[t] - m * sc)

        def chunk(c, carry):
            base = c * FCHUNK
            acc = jnp.zeros((FCHUNK, DO), jnp.float32) + bp_ref[0]
            for t in range(N_EXP):
                z = z2_ref[pl.ds(t * NP + base, FCHUNK), :]
                p = jnp.maximum(z * scales[t] + shifts[t], 0.0)
                acc = acc + wp_ref[t] * p
            px_out_ref[pl.ds(base, FCHUNK), :] = acc
            return carry

        jax.lax.fori_loop(0, NP // FCHUNK, chunk, 0, unroll=False)


@functools.partial(jax.jit, static_argnames=("interpret",))
def _run(px, lx, pl_mat_list, W_pl, b_pl, g_pl, be_pl,
         W_lp, b_lp, g_lp, be_lp, wp3, bp3, wl3, bl3, interpret=False):
    term_blk = lambda i, r: (i, 0, 0)
    const2 = lambda i, r: (0, 0)
    const3 = lambda i, r: (0, 0, 0)
    vec_spec = pl.BlockSpec((1, 1, DO), term_blk)
    full_vec_spec = pl.BlockSpec((N_EXP, 1, DO), const3)
    return pl.pallas_call(
        _body,
        grid=(N_EXP, NR),
        in_specs=[
            pl.BlockSpec(memory_space=pl.ANY),
            pl.BlockSpec((BLK, DP), lambda i, r: (r, 0)),
            pl.BlockSpec((NL, DL), const2),
            pl.BlockSpec((1, DIN, DO), term_blk),
            vec_spec,
            full_vec_spec,
            full_vec_spec,
            pl.BlockSpec((1, DIN, DO), term_blk),
            vec_spec,
            vec_spec,
            vec_spec,
            pl.BlockSpec((N_EXP, 1, 1), const3),
            pl.BlockSpec((1, 1, 1), const3),
            pl.BlockSpec((1, 1, 1), term_blk),
            pl.BlockSpec((1, 1, 1), const3),
        ],
        out_specs=[
            pl.BlockSpec((NP, DO), const2),
            pl.BlockSpec((NL, DO), const2),
        ],
        out_shape=[
            jax.ShapeDtypeStruct((NP, DO), jnp.float32),
            jax.ShapeDtypeStruct((NL, DO), jnp.float32),
        ],
        scratch_shapes=[
            pltpu.VMEM((N_EXP * NP, DO), jnp.float32),
            pltpu.VMEM((DO, NL), jnp.float32),
            pltpu.VMEM((NL, DO), jnp.float32),
            pltpu.VMEM((8, DO), jnp.float32),
            pltpu.VMEM((8, DO), jnp.float32),
            pltpu.VMEM((2, BLK, NL), jnp.float32),
            pltpu.SemaphoreType.DMA((2,)),
        ],
        compiler_params=pltpu.CompilerParams(
            dimension_semantics=("arbitrary", "arbitrary"),
        ),
        interpret=interpret,
    )(pl_mat_list.reshape(N_EXP * NP, NL), px, lx, W_pl, b_pl, g_pl, be_pl,
      W_lp, b_lp, g_lp, be_lp, wp3, bp3, wl3, bl3)


def kernel(px, lx, pl_mat_list, W_pl, b_pl, g_pl, be_pl,
           W_lp, b_lp, g_lp, be_lp, wp, bp, wl, bl):
    v = lambda a: a.reshape(N_EXP, 1, DO)
    px_out, lx_out = _run(px, lx, pl_mat_list,
                          W_pl, v(b_pl), v(g_pl), v(be_pl),
                          W_lp, v(b_lp), v(g_lp), v(be_lp),
                          wp.reshape(N_EXP, 1, 1), bp.reshape(1, 1, 1),
                          wl.reshape(N_EXP, 1, 1), bl.reshape(1, 1, 1))
    return (px_out, lx_out)


# speedup vs baseline: 1.0571x; 1.0275x over previous
import functools

import jax
import jax.numpy as jnp
from jax.experimental import pallas as pl
from jax.experimental.pallas import tpu as pltpu

N_EXP = 4
NP = 8192
NL = 2048
DP = 64
DL = 64
DO = 64
DIN = DP + DL
EPS = 1e-5

BLK = 1024
NR = NP // BLK
FCHUNK = 1024


def _body(am_hbm_ref, px_ref, lx_ref,
          Wpl_ref, bpl_ref, gpl_ref, bepl_ref,
          Wlp_ref, blp_ref, glp_ref, belp_ref,
          wp_ref, bp_ref, wl_ref, bl_ref, Wcat_ref,
          px_out_ref, lx_out_ref,
          z2_ref, x1T_ref, Li_ref, s_ref, q_ref, am_buf, dma_sem):
    i = pl.program_id(0)
    r = pl.program_id(1)
    nr = pl.num_programs(1)
    k = i * nr + r
    nk = pl.num_programs(0) * nr

    def start_copy(kk):
        slot = jax.lax.rem(kk, 2)
        pltpu.make_async_copy(
            am_hbm_ref.at[pl.ds(kk * BLK, BLK), :],
            am_buf.at[slot], dma_sem.at[slot]).start()

    @pl.when(k == 0)
    def _():
        start_copy(0)

    @pl.when(k + 1 < nk)
    def _():
        start_copy(k + 1)

    slot = jax.lax.rem(k, 2)
    pltpu.make_async_copy(
        am_hbm_ref.at[pl.ds(k * BLK, BLK), :],
        am_buf.at[slot], dma_sem.at[slot]).wait()

    am = am_buf[slot]
    pxb = px_ref[...]

    @pl.when(r == 0)
    def _():
        Li_ref[...] = jnp.dot(lx_ref[...], Wpl_ref[0][:DL],
                              preferred_element_type=jnp.float32)

    both = jnp.dot(pxb, Wcat_ref[0], preferred_element_type=jnp.float32)

    z2 = (jnp.dot(am, Li_ref[...], preferred_element_type=jnp.float32)
          + both[:, :DO] + bpl_ref[0])
    z2_ref[pl.ds(i * NP + r * BLK, BLK), :] = z2

    s_blk = jnp.sum(z2, axis=0, keepdims=True)
    q_blk = jnp.sum(z2 * z2, axis=0, keepdims=True)

    @pl.when(r == 0)
    def _():
        s_ref[pl.ds(i, 1), :] = s_blk
        q_ref[pl.ds(i, 1), :] = q_blk

    @pl.when(r != 0)
    def _():
        s_ref[pl.ds(i, 1), :] = s_ref[pl.ds(i, 1), :] + s_blk
        q_ref[pl.ds(i, 1), :] = q_ref[pl.ds(i, 1), :] + q_blk

    pxw = both[:, DO:]
    partT = jax.lax.dot_general(pxw, am, (((0,), (0,)), ((), ())),
                                preferred_element_type=jnp.float32)

    @pl.when(r == 0)
    def _():
        x1T_ref[...] = partT

    @pl.when(r != 0)
    def _():
        x1T_ref[...] = x1T_ref[...] + partT

    @pl.when(r == nr - 1)
    def _():
        z1 = (x1T_ref[...].T
              + jnp.dot(lx_ref[...], Wlp_ref[0][DP:],
                        preferred_element_type=jnp.float32)
              + blp_ref[0])
        m1 = jnp.mean(z1, axis=0, keepdims=True)
        v1 = jnp.mean(z1 * z1, axis=0, keepdims=True) - m1 * m1
        s1 = glp_ref[0] * jax.lax.rsqrt(v1 + EPS)
        p1 = jnp.maximum(z1 * s1 + (belp_ref[0] - m1 * s1), 0.0)
        contrib_l = wl_ref[0] * p1

        @pl.when(i == 0)
        def _():
            lx_out_ref[...] = contrib_l + bl_ref[0]

        @pl.when(i != 0)
        def _():
            lx_out_ref[...] = lx_out_ref[...] + contrib_l

    @pl.when((i == pl.num_programs(0) - 1) & (r == nr - 1))
    def _():
        scales = []
        shifts = []
        for t in range(N_EXP):
            m = s_ref[t:t + 1, :] * (1.0 / NP)
            v = q_ref[t:t + 1, :] * (1.0 / NP) - m * m
            sc = gpl_ref[t] * jax.lax.rsqrt(v + EPS)
            scales.append(sc)
            shifts.append(bepl_ref[t] - m * sc)

        def chunk(c, carry):
            base = c * FCHUNK
            acc = jnp.zeros((FCHUNK, DO), jnp.float32) + bp_ref[0]
            for t in range(N_EXP):
                z = z2_ref[pl.ds(t * NP + base, FCHUNK), :]
                p = jnp.maximum(z * scales[t] + shifts[t], 0.0)
                acc = acc + wp_ref[t] * p
            px_out_ref[pl.ds(base, FCHUNK), :] = acc
            return carry

        jax.lax.fori_loop(0, NP // FCHUNK, chunk, 0, unroll=False)


@functools.partial(jax.jit, static_argnames=("interpret",))
def _run(px, lx, pl_mat_list, W_pl, b_pl, g_pl, be_pl,
         W_lp, b_lp, g_lp, be_lp, wp3, bp3, wl3, bl3, interpret=False):
    term_blk = lambda i, r: (i, 0, 0)
    const2 = lambda i, r: (0, 0)
    const3 = lambda i, r: (0, 0, 0)
    vec_spec = pl.BlockSpec((1, 1, DO), term_blk)
    full_vec_spec = pl.BlockSpec((N_EXP, 1, DO), const3)
    _call = pl.pallas_call(
        _body,
        grid=(N_EXP, NR),
        in_specs=[
            pl.BlockSpec(memory_space=pl.ANY),
            pl.BlockSpec((BLK, DP), lambda i, r: (r, 0)),
            pl.BlockSpec((NL, DL), const2),
            pl.BlockSpec((1, DIN, DO), term_blk),
            vec_spec,
            full_vec_spec,
            full_vec_spec,
            pl.BlockSpec((1, DIN, DO), term_blk),
            vec_spec,
            vec_spec,
            vec_spec,
            pl.BlockSpec((N_EXP, 1, 1), const3),
            pl.BlockSpec((1, 1, 1), const3),
            pl.BlockSpec((1, 1, 1), term_blk),
            pl.BlockSpec((1, 1, 1), const3),
            pl.BlockSpec((1, DP, 2 * DO), term_blk),
        ],
        out_specs=[
            pl.BlockSpec((NP, DO), const2),
            pl.BlockSpec((NL, DO), const2),
        ],
        out_shape=[
            jax.ShapeDtypeStruct((NP, DO), jnp.float32),
            jax.ShapeDtypeStruct((NL, DO), jnp.float32),
        ],
        scratch_shapes=[
            pltpu.VMEM((N_EXP * NP, DO), jnp.float32),
            pltpu.VMEM((DO, NL), jnp.float32),
            pltpu.VMEM((NL, DO), jnp.float32),
            pltpu.VMEM((8, DO), jnp.float32),
            pltpu.VMEM((8, DO), jnp.float32),
            pltpu.VMEM((2, BLK, NL), jnp.float32),
            pltpu.SemaphoreType.DMA((2,)),
        ],
        compiler_params=pltpu.CompilerParams(
            dimension_semantics=("arbitrary", "arbitrary"),
        ),
        interpret=interpret,
    )
    Wcat = jnp.concatenate([W_pl[:, DL:, :], W_lp[:, :DP, :]], axis=2)
    return _call(pl_mat_list.reshape(N_EXP * NP, NL), px, lx,
                 W_pl, b_pl, g_pl, be_pl,
                 W_lp, b_lp, g_lp, be_lp, wp3, bp3, wl3, bl3, Wcat)


def kernel(px, lx, pl_mat_list, W_pl, b_pl, g_pl, be_pl,
           W_lp, b_lp, g_lp, be_lp, wp, bp, wl, bl):
    v = lambda a: a.reshape(N_EXP, 1, DO)
    px_out, lx_out = _run(px, lx, pl_mat_list,
                          W_pl, v(b_pl), v(g_pl), v(be_pl),
                          W_lp, v(b_lp), v(g_lp), v(be_lp),
                          wp.reshape(N_EXP, 1, 1), bp.reshape(1, 1, 1),
                          wl.reshape(N_EXP, 1, 1), bl.reshape(1, 1, 1))
    return (px_out, lx_out)


# speedup vs baseline: 1.0607x; 1.0034x over previous
import functools

import jax
import jax.numpy as jnp
from jax.experimental import pallas as pl
from jax.experimental.pallas import tpu as pltpu

N_EXP = 4
NP = 8192
NL = 2048
DP = 64
DL = 64
DO = 64
DIN = DP + DL
EPS = 1e-5

BLK = 1024
NR = NP // BLK
FCHUNK = 1024


def _body(am_hbm_ref, px_ref, lx_ref,
          Wpl_ref, bpl_ref, gpl_ref, bepl_ref,
          Wlp_ref, blp_ref, glp_ref, belp_ref,
          wp_ref, bp_ref, wl_ref, bl_ref, Wcat_ref,
          px_out_ref, lx_out_ref,
          z2_ref, x1T_ref, Li_ref, s_ref, q_ref, am_buf, dma_sem):
    i = pl.program_id(0)
    r = pl.program_id(1)
    nr = pl.num_programs(1)
    k = i * nr + r
    nk = pl.num_programs(0) * nr

    def start_copy(kk):
        slot = jax.lax.rem(kk, 2)
        pltpu.make_async_copy(
            am_hbm_ref.at[pl.ds(kk * BLK, BLK), :],
            am_buf.at[slot], dma_sem.at[slot]).start()

    @pl.when(k == 0)
    def _():
        start_copy(0)

    @pl.when(k + 1 < nk)
    def _():
        start_copy(k + 1)

    slot = jax.lax.rem(k, 2)
    pltpu.make_async_copy(
        am_hbm_ref.at[pl.ds(k * BLK, BLK), :],
        am_buf.at[slot], dma_sem.at[slot]).wait()

    am = am_buf[slot]
    pxb = px_ref[pl.ds(r * BLK, BLK), :]

    @pl.when(r == 0)
    def _():
        Li_ref[...] = jnp.dot(lx_ref[...], Wpl_ref[0][:DL],
                              preferred_element_type=jnp.float32)

    both = jnp.dot(pxb, Wcat_ref[0], preferred_element_type=jnp.float32)

    z2 = (jnp.dot(am, Li_ref[...], preferred_element_type=jnp.float32)
          + both[:, :DO] + bpl_ref[0])
    z2_ref[pl.ds(i * NP + r * BLK, BLK), :] = z2

    s_blk = jnp.sum(z2, axis=0, keepdims=True)
    q_blk = jnp.sum(z2 * z2, axis=0, keepdims=True)

    @pl.when(r == 0)
    def _():
        s_ref[pl.ds(i, 1), :] = s_blk
        q_ref[pl.ds(i, 1), :] = q_blk

    @pl.when(r != 0)
    def _():
        s_ref[pl.ds(i, 1), :] = s_ref[pl.ds(i, 1), :] + s_blk
        q_ref[pl.ds(i, 1), :] = q_ref[pl.ds(i, 1), :] + q_blk

    pxw = both[:, DO:]
    partT = jax.lax.dot_general(pxw, am, (((0,), (0,)), ((), ())),
                                preferred_element_type=jnp.float32)

    @pl.when(r == 0)
    def _():
        x1T_ref[...] = partT

    @pl.when(r != 0)
    def _():
        x1T_ref[...] = x1T_ref[...] + partT

    @pl.when(r == nr - 1)
    def _():
        z1 = (x1T_ref[...].T
              + jnp.dot(lx_ref[...], Wlp_ref[0][DP:],
                        preferred_element_type=jnp.float32)
              + blp_ref[0])
        m1 = jnp.mean(z1, axis=0, keepdims=True)
        v1 = jnp.mean(z1 * z1, axis=0, keepdims=True) - m1 * m1
        s1 = glp_ref[0] * jax.lax.rsqrt(v1 + EPS)
        p1 = jnp.maximum(z1 * s1 + (belp_ref[0] - m1 * s1), 0.0)
        contrib_l = wl_ref[0] * p1

        @pl.when(i == 0)
        def _():
            lx_out_ref[...] = contrib_l + bl_ref[0]

        @pl.when(i != 0)
        def _():
            lx_out_ref[...] = lx_out_ref[...] + contrib_l

    @pl.when((i == pl.num_programs(0) - 1) & (r == nr - 1))
    def _():
        scales = []
        shifts = []
        for t in range(N_EXP):
            m = s_ref[t:t + 1, :] * (1.0 / NP)
            v = q_ref[t:t + 1, :] * (1.0 / NP) - m * m
            sc = gpl_ref[t] * jax.lax.rsqrt(v + EPS)
            scales.append(sc)
            shifts.append(bepl_ref[t] - m * sc)

        def chunk(c, carry):
            base = c * FCHUNK
            acc = jnp.zeros((FCHUNK, DO), jnp.float32) + bp_ref[0]
            for t in range(N_EXP):
                z = z2_ref[pl.ds(t * NP + base, FCHUNK), :]
                p = jnp.maximum(z * scales[t] + shifts[t], 0.0)
                acc = acc + wp_ref[t] * p
            px_out_ref[pl.ds(base, FCHUNK), :] = acc
            return carry

        jax.lax.fori_loop(0, NP // FCHUNK, chunk, 0, unroll=False)


@functools.partial(jax.jit, static_argnames=("interpret",))
def _run(px, lx, pl_mat_list, W_pl, b_pl, g_pl, be_pl,
         W_lp, b_lp, g_lp, be_lp, wp3, bp3, wl3, bl3, interpret=False):
    term_blk = lambda i, r: (i, 0, 0)
    const2 = lambda i, r: (0, 0)
    const3 = lambda i, r: (0, 0, 0)
    vec_spec = pl.BlockSpec((1, 1, DO), term_blk)
    full_vec_spec = pl.BlockSpec((N_EXP, 1, DO), const3)
    _call = pl.pallas_call(
        _body,
        grid=(N_EXP, NR),
        in_specs=[
            pl.BlockSpec(memory_space=pl.ANY),
            pl.BlockSpec((NP, DP), const2),
            pl.BlockSpec((NL, DL), const2),
            pl.BlockSpec((1, DIN, DO), term_blk),
            vec_spec,
            full_vec_spec,
            full_vec_spec,
            pl.BlockSpec((1, DIN, DO), term_blk),
            vec_spec,
            vec_spec,
            vec_spec,
            pl.BlockSpec((N_EXP, 1, 1), const3),
            pl.BlockSpec((1, 1, 1), const3),
            pl.BlockSpec((1, 1, 1), term_blk),
            pl.BlockSpec((1, 1, 1), const3),
            pl.BlockSpec((1, DP, 2 * DO), term_blk),
        ],
        out_specs=[
            pl.BlockSpec((NP, DO), const2),
            pl.BlockSpec((NL, DO), const2),
        ],
        out_shape=[
            jax.ShapeDtypeStruct((NP, DO), jnp.float32),
            jax.ShapeDtypeStruct((NL, DO), jnp.float32),
        ],
        scratch_shapes=[
            pltpu.VMEM((N_EXP * NP, DO), jnp.float32),
            pltpu.VMEM((DO, NL), jnp.float32),
            pltpu.VMEM((NL, DO), jnp.float32),
            pltpu.VMEM((8, DO), jnp.float32),
            pltpu.VMEM((8, DO), jnp.float32),
            pltpu.VMEM((2, BLK, NL), jnp.float32),
            pltpu.SemaphoreType.DMA((2,)),
        ],
        compiler_params=pltpu.CompilerParams(
            dimension_semantics=("arbitrary", "arbitrary"),
        ),
        interpret=interpret,
    )
    Wcat = jnp.concatenate([W_pl[:, DL:, :], W_lp[:, :DP, :]], axis=2)
    return _call(pl_mat_list.reshape(N_EXP * NP, NL), px, lx,
                 W_pl, b_pl, g_pl, be_pl,
                 W_lp, b_lp, g_lp, be_lp, wp3, bp3, wl3, bl3, Wcat)


def kernel(px, lx, pl_mat_list, W_pl, b_pl, g_pl, be_pl,
           W_lp, b_lp, g_lp, be_lp, wp, bp, wl, bl):
    v = lambda a: a.reshape(N_EXP, 1, DO)
    px_out, lx_out = _run(px, lx, pl_mat_list,
                          W_pl, v(b_pl), v(g_pl), v(be_pl),
                          W_lp, v(b_lp), v(g_lp), v(be_lp),
                          wp.reshape(N_EXP, 1, 1), bp.reshape(1, 1, 1),
                          wl.reshape(N_EXP, 1, 1), bl.reshape(1, 1, 1))
    return (px_out, lx_out)


# speedup vs baseline: 1.1314x; 1.0667x over previous
import functools

import jax
import jax.numpy as jnp
from jax.experimental import pallas as pl
from jax.experimental.pallas import tpu as pltpu

N_EXP = 4
NP = 8192
NL = 2048
DP = 64
DL = 64
DO = 64
DIN = DP + DL
EPS = 1e-5

BLK = 1024
NR = NP // BLK
FCHUNK = 1024


def _body(am_hbm_ref, px_ref, lx_ref,
          Wpl_ref, bpl_ref, gpl_ref, bepl_ref,
          Wlp_ref, blp_ref, glp_ref, belp_ref,
          wp_ref, bp_ref, wl_ref, bl_ref, Wcat_ref,
          px_out_ref, lx_out_ref,
          z2_ref, x1T_ref, Li_ref, s_ref, q_ref, am_buf, dma_sem):
    i = pl.program_id(0)
    r = pl.program_id(1)
    nr = pl.num_programs(1)
    k = i * nr + r
    nk = pl.num_programs(0) * nr

    def start_copy(kk):
        slot = jax.lax.rem(kk, 2)
        pltpu.make_async_copy(
            am_hbm_ref.at[pl.ds(kk * BLK, BLK), :],
            am_buf.at[slot], dma_sem.at[slot]).start()

    @pl.when(k == 0)
    def _():
        start_copy(0)

    @pl.when(k + 1 < nk)
    def _():
        start_copy(k + 1)

    slot = jax.lax.rem(k, 2)
    pltpu.make_async_copy(
        am_hbm_ref.at[pl.ds(k * BLK, BLK), :],
        am_buf.at[slot], dma_sem.at[slot]).wait()

    pxb = px_ref[pl.ds(r * BLK, BLK), :]

    @pl.when(r == 0)
    def _():
        Li_ref[...] = jnp.dot(lx_ref[...], Wpl_ref[0][:DL],
                              preferred_element_type=jnp.float32)

    both = jnp.dot(pxb, Wcat_ref[0], preferred_element_type=jnp.float32)

    z2 = (jnp.dot(am_buf[slot], Li_ref[...],
                  preferred_element_type=jnp.float32)
          + both[:, :DO] + bpl_ref[0])
    z2_ref[pl.ds(i * NP + r * BLK, BLK), :] = z2

    s_blk = jnp.sum(z2, axis=0, keepdims=True)
    q_blk = jnp.sum(z2 * z2, axis=0, keepdims=True)

    @pl.when(r == 0)
    def _():
        s_ref[pl.ds(i, 1), :] = s_blk
        q_ref[pl.ds(i, 1), :] = q_blk

    @pl.when(r != 0)
    def _():
        s_ref[pl.ds(i, 1), :] = s_ref[pl.ds(i, 1), :] + s_blk
        q_ref[pl.ds(i, 1), :] = q_ref[pl.ds(i, 1), :] + q_blk

    pxw = both[:, DO:]
    partT = jax.lax.dot_general(pxw, am_buf[slot], (((0,), (0,)), ((), ())),
                                preferred_element_type=jnp.float32)

    @pl.when(r == 0)
    def _():
        x1T_ref[...] = partT

    @pl.when(r != 0)
    def _():
        x1T_ref[...] = x1T_ref[...] + partT

    @pl.when(r == nr - 1)
    def _():
        z1 = (x1T_ref[...].T
              + jnp.dot(lx_ref[...], Wlp_ref[0][DP:],
                        preferred_element_type=jnp.float32)
              + blp_ref[0])
        m1 = jnp.mean(z1, axis=0, keepdims=True)
        v1 = jnp.mean(z1 * z1, axis=0, keepdims=True) - m1 * m1
        s1 = glp_ref[0] * jax.lax.rsqrt(v1 + EPS)
        p1 = jnp.maximum(z1 * s1 + (belp_ref[0] - m1 * s1), 0.0)
        contrib_l = wl_ref[0] * p1

        @pl.when(i == 0)
        def _():
            lx_out_ref[...] = contrib_l + bl_ref[0]

        @pl.when(i != 0)
        def _():
            lx_out_ref[...] = lx_out_ref[...] + contrib_l

    @pl.when((i == pl.num_programs(0) - 1) & (r == nr - 1))
    def _():
        scales = []
        shifts = []
        for t in range(N_EXP):
            m = s_ref[t:t + 1, :] * (1.0 / NP)
            v = q_ref[t:t + 1, :] * (1.0 / NP) - m * m
            sc = gpl_ref[t] * jax.lax.rsqrt(v + EPS)
            scales.append(sc)
            shifts.append(bepl_ref[t] - m * sc)

        def chunk(c, carry):
            base = c * FCHUNK
            acc = jnp.zeros((FCHUNK, DO), jnp.float32) + bp_ref[0]
            for t in range(N_EXP):
                z = z2_ref[pl.ds(t * NP + base, FCHUNK), :]
                p = jnp.maximum(z * scales[t] + shifts[t], 0.0)
                acc = acc + wp_ref[t] * p
            px_out_ref[pl.ds(base, FCHUNK), :] = acc
            return carry

        jax.lax.fori_loop(0, NP // FCHUNK, chunk, 0, unroll=False)


@functools.partial(jax.jit, static_argnames=("interpret",))
def _run(px, lx, pl_mat_list, W_pl, b_pl, g_pl, be_pl,
         W_lp, b_lp, g_lp, be_lp, wp3, bp3, wl3, bl3, interpret=False):
    term_blk = lambda i, r: (i, 0, 0)
    const2 = lambda i, r: (0, 0)
    const3 = lambda i, r: (0, 0, 0)
    vec_spec = pl.BlockSpec((1, 1, DO), term_blk)
    full_vec_spec = pl.BlockSpec((N_EXP, 1, DO), const3)
    _call = pl.pallas_call(
        _body,
        grid=(N_EXP, NR),
        in_specs=[
            pl.BlockSpec(memory_space=pl.ANY),
            pl.BlockSpec((NP, DP), const2),
            pl.BlockSpec((NL, DL), const2),
            pl.BlockSpec((1, DIN, DO), term_blk),
            vec_spec,
            full_vec_spec,
            full_vec_spec,
            pl.BlockSpec((1, DIN, DO), term_blk),
            vec_spec,
            vec_spec,
            vec_spec,
            pl.BlockSpec((N_EXP, 1, 1), const3),
            pl.BlockSpec((1, 1, 1), const3),
            pl.BlockSpec((1, 1, 1), term_blk),
            pl.BlockSpec((1, 1, 1), const3),
            pl.BlockSpec((1, DP, 2 * DO), term_blk),
        ],
        out_specs=[
            pl.BlockSpec((NP, DO), const2),
            pl.BlockSpec((NL, DO), const2),
        ],
        out_shape=[
            jax.ShapeDtypeStruct((NP, DO), jnp.float32),
            jax.ShapeDtypeStruct((NL, DO), jnp.float32),
        ],
        scratch_shapes=[
            pltpu.VMEM((N_EXP * NP, DO), jnp.float32),
            pltpu.VMEM((DO, NL), jnp.float32),
            pltpu.VMEM((NL, DO), jnp.float32),
            pltpu.VMEM((8, DO), jnp.float32),
            pltpu.VMEM((8, DO), jnp.float32),
            pltpu.VMEM((2, BLK, NL), jnp.float32),
            pltpu.SemaphoreType.DMA((2,)),
        ],
        compiler_params=pltpu.CompilerParams(
            dimension_semantics=("arbitrary", "arbitrary"),
        ),
        interpret=interpret,
    )
    Wcat = jnp.concatenate([W_pl[:, DL:, :], W_lp[:, :DP, :]], axis=2)
    return _call(pl_mat_list.reshape(N_EXP * NP, NL), px, lx,
                 W_pl, b_pl, g_pl, be_pl,
                 W_lp, b_lp, g_lp, be_lp, wp3, bp3, wl3, bl3, Wcat)


def kernel(px, lx, pl_mat_list, W_pl, b_pl, g_pl, be_pl,
           W_lp, b_lp, g_lp, be_lp, wp, bp, wl, bl):
    v = lambda a: a.reshape(N_EXP, 1, DO)
    px_out, lx_out = _run(px, lx, pl_mat_list,
                          W_pl, v(b_pl), v(g_pl), v(be_pl),
                          W_lp, v(b_lp), v(g_lp), v(be_lp),
                          wp.reshape(N_EXP, 1, 1), bp.reshape(1, 1, 1),
                          wl.reshape(N_EXP, 1, 1), bl.reshape(1, 1, 1))
    return (px_out, lx_out)


# speedup vs baseline: 1.2336x; 1.0903x over previous
import functools

import jax
import jax.numpy as jnp
from jax.experimental import pallas as pl
from jax.experimental.pallas import tpu as pltpu

N_EXP = 4
NP = 8192
NL = 2048
DP = 64
DL = 64
DO = 64
DIN = DP + DL
EPS = 1e-5

BLK = 1024
NR = NP // BLK
FCHUNK = 1024


def _body(am_hbm_ref, px_ref, lx_ref,
          Wpl_ref, bpl_ref, gpl_ref, bepl_ref,
          Wlp_ref, blp_ref, glp_ref, belp_ref,
          wp_ref, bp_ref, wl_ref, bl_ref, Wcat_ref,
          px_out_ref, lx_out_ref,
          z2_ref, x1T_ref, Li_ref, s_ref, q_ref, am_buf, dma_sem):
    i = pl.program_id(0)
    r = pl.program_id(1)
    nr = pl.num_programs(1)
    k = i * nr + r
    nk = pl.num_programs(0) * nr

    def start_copy(kk):
        slot = jax.lax.rem(kk, 3)
        pltpu.make_async_copy(
            am_hbm_ref.at[pl.ds(kk * BLK, BLK), :],
            am_buf.at[slot], dma_sem.at[slot]).start()

    @pl.when(k == 0)
    def _():
        start_copy(0)
        start_copy(1)

    @pl.when((k + 2 < nk))
    def _():
        start_copy(k + 2)

    slot = jax.lax.rem(k, 3)
    pltpu.make_async_copy(
        am_hbm_ref.at[pl.ds(k * BLK, BLK), :],
        am_buf.at[slot], dma_sem.at[slot]).wait()

    pxb = px_ref[pl.ds(r * BLK, BLK), :]

    @pl.when(r == 0)
    def _():
        Li_ref[...] = jnp.dot(lx_ref[...], Wpl_ref[0][:DL],
                              preferred_element_type=jnp.float32)

    both = jnp.dot(pxb, Wcat_ref[0], preferred_element_type=jnp.float32)

    z2 = (jnp.dot(am_buf[slot], Li_ref[...],
                  preferred_element_type=jnp.float32)
          + both[:, :DO] + bpl_ref[0])
    z2_ref[pl.ds(i * NP + r * BLK, BLK), :] = z2

    s_blk = jnp.sum(z2, axis=0, keepdims=True)
    q_blk = jnp.sum(z2 * z2, axis=0, keepdims=True)

    @pl.when(r == 0)
    def _():
        s_ref[pl.ds(i, 1), :] = s_blk
        q_ref[pl.ds(i, 1), :] = q_blk

    @pl.when(r != 0)
    def _():
        s_ref[pl.ds(i, 1), :] = s_ref[pl.ds(i, 1), :] + s_blk
        q_ref[pl.ds(i, 1), :] = q_ref[pl.ds(i, 1), :] + q_blk

    pxw = both[:, DO:]
    partT = jax.lax.dot_general(pxw, am_buf[slot], (((0,), (0,)), ((), ())),
                                preferred_element_type=jnp.float32)

    @pl.when(r == 0)
    def _():
        x1T_ref[...] = partT

    @pl.when(r != 0)
    def _():
        x1T_ref[...] = x1T_ref[...] + partT

    @pl.when(r == nr - 1)
    def _():
        z1 = (x1T_ref[...].T
              + jnp.dot(lx_ref[...], Wlp_ref[0][DP:],
                        preferred_element_type=jnp.float32)
              + blp_ref[0])
        m1 = jnp.mean(z1, axis=0, keepdims=True)
        v1 = jnp.mean(z1 * z1, axis=0, keepdims=True) - m1 * m1
        s1 = glp_ref[0] * jax.lax.rsqrt(v1 + EPS)
        p1 = jnp.maximum(z1 * s1 + (belp_ref[0] - m1 * s1), 0.0)
        contrib_l = wl_ref[0] * p1

        @pl.when(i == 0)
        def _():
            lx_out_ref[...] = contrib_l + bl_ref[0]

        @pl.when(i != 0)
        def _():
            lx_out_ref[...] = lx_out_ref[...] + contrib_l

    @pl.when((i == pl.num_programs(0) - 1) & (r == nr - 1))
    def _():
        scales = []
        shifts = []
        for t in range(N_EXP):
            m = s_ref[t:t + 1, :] * (1.0 / NP)
            v = q_ref[t:t + 1, :] * (1.0 / NP) - m * m
            sc = gpl_ref[t] * jax.lax.rsqrt(v + EPS)
            scales.append(sc)
            shifts.append(bepl_ref[t] - m * sc)

        def chunk(c, carry):
            base = c * FCHUNK
            acc = jnp.zeros((FCHUNK, DO), jnp.float32) + bp_ref[0]
            for t in range(N_EXP):
                z = z2_ref[pl.ds(t * NP + base, FCHUNK), :]
                p = jnp.maximum(z * scales[t] + shifts[t], 0.0)
                acc = acc + wp_ref[t] * p
            px_out_ref[pl.ds(base, FCHUNK), :] = acc
            return carry

        jax.lax.fori_loop(0, NP // FCHUNK, chunk, 0, unroll=False)


@functools.partial(jax.jit, static_argnames=("interpret",))
def _run(px, lx, pl_mat_list, W_pl, b_pl, g_pl, be_pl,
         W_lp, b_lp, g_lp, be_lp, wp3, bp3, wl3, bl3, interpret=False):
    term_blk = lambda i, r: (i, 0, 0)
    const2 = lambda i, r: (0, 0)
    const3 = lambda i, r: (0, 0, 0)
    vec_spec = pl.BlockSpec((1, 1, DO), term_blk)
    full_vec_spec = pl.BlockSpec((N_EXP, 1, DO), const3)
    _call = pl.pallas_call(
        _body,
        grid=(N_EXP, NR),
        in_specs=[
            pl.BlockSpec(memory_space=pl.ANY),
            pl.BlockSpec((NP, DP), const2),
            pl.BlockSpec((NL, DL), const2),
            pl.BlockSpec((1, DIN, DO), term_blk),
            vec_spec,
            full_vec_spec,
            full_vec_spec,
            pl.BlockSpec((1, DIN, DO), term_blk),
            vec_spec,
            vec_spec,
            vec_spec,
            pl.BlockSpec((N_EXP, 1, 1), const3),
            pl.BlockSpec((1, 1, 1), const3),
            pl.BlockSpec((1, 1, 1), term_blk),
            pl.BlockSpec((1, 1, 1), const3),
            pl.BlockSpec((1, DP, 2 * DO), term_blk),
        ],
        out_specs=[
            pl.BlockSpec((NP, DO), const2),
            pl.BlockSpec((NL, DO), const2),
        ],
        out_shape=[
            jax.ShapeDtypeStruct((NP, DO), jnp.float32),
            jax.ShapeDtypeStruct((NL, DO), jnp.float32),
        ],
        scratch_shapes=[
            pltpu.VMEM((N_EXP * NP, DO), jnp.float32),
            pltpu.VMEM((DO, NL), jnp.float32),
            pltpu.VMEM((NL, DO), jnp.float32),
            pltpu.VMEM((8, DO), jnp.float32),
            pltpu.VMEM((8, DO), jnp.float32),
            pltpu.VMEM((3, BLK, NL), jnp.float32),
            pltpu.SemaphoreType.DMA((3,)),
        ],
        compiler_params=pltpu.CompilerParams(
            dimension_semantics=("arbitrary", "arbitrary"),
        ),
        interpret=interpret,
    )
    Wcat = jnp.concatenate([W_pl[:, DL:, :], W_lp[:, :DP, :]], axis=2)
    return _call(pl_mat_list.reshape(N_EXP * NP, NL), px, lx,
                 W_pl, b_pl, g_pl, be_pl,
                 W_lp, b_lp, g_lp, be_lp, wp3, bp3, wl3, bl3, Wcat)


def kernel(px, lx, pl_mat_list, W_pl, b_pl, g_pl, be_pl,
           W_lp, b_lp, g_lp, be_lp, wp, bp, wl, bl):
    v = lambda a: a.reshape(N_EXP, 1, DO)
    px_out, lx_out = _run(px, lx, pl_mat_list,
                          W_pl, v(b_pl), v(g_pl), v(be_pl),
                          W_lp, v(b_lp), v(g_lp), v(be_lp),
                          wp.reshape(N_EXP, 1, 1), bp.reshape(1, 1, 1),
                          wl.reshape(N_EXP, 1, 1), bl.reshape(1, 1, 1))
    return (px_out, lx_out)
